# Initial kernel scaffold; baseline (speedup 1.0000x reference)
import jax
import jax.numpy as jnp
from jax.experimental import pallas as pl


def kernel(indices, pe):
    raise NotImplementedError("write your pallas kernel here")


# speedup vs baseline: 7.9898x; 7.9898x over previous
import functools

import jax
import jax.numpy as jnp
from jax import lax
from jax.experimental import pallas as pl
from jax.experimental.pallas import tpu as pltpu
from jax.experimental.pallas import tpu_sc as plsc

B = 4096
K = 200
D = 128
NC = 2
NS = 16
NW = NC * NS
BPW = B // NW

_mesh = plsc.VectorSubcoreMesh(core_axis_name="c", subcore_axis_name="s")


@functools.partial(
    pl.kernel,
    mesh=_mesh,
    out_type=jax.ShapeDtypeStruct((B, K, D), jnp.float32),
    scratch_types=[
        pltpu.VMEM((BPW * K,), jnp.int32),
        pltpu.VMEM((K, D), jnp.float32),
        pltpu.SemaphoreType.DMA,
    ],
)
def _gather_pe(table_hbm, idx_hbm, out_hbm, idx_v, rows_v, gsem):
    wid = lax.axis_index("s") * NC + lax.axis_index("c")
    b0 = wid * BPW
    pltpu.sync_copy(idx_hbm.at[pl.ds(b0 * K, BPW * K)], idx_v)

    def body(i, carry):
        pltpu.async_copy(
            table_hbm.at[idx_v.at[pl.ds(i * K, K)]], rows_v, gsem
        ).wait()
        pltpu.sync_copy(rows_v, out_hbm.at[b0 + i])
        return carry

    lax.fori_loop(0, BPW, body, 0)


def kernel(indices, pe):
    table = pe[0]
    idx = indices.astype(jnp.int32).reshape(-1)
    return _gather_pe(table, idx)


# speedup vs baseline: 9.4641x; 1.1845x over previous
import functools

import jax
import jax.numpy as jnp
from jax import lax
from jax.experimental import pallas as pl
from jax.experimental.pallas import tpu as pltpu
from jax.experimental.pallas import tpu_sc as plsc

B = 4096
K = 200
D = 128
V = 8192
NC = 2
NS = 16
NW = NC * NS
BPW = B // NW
VPT = V // NS

_mesh = plsc.VectorSubcoreMesh(core_axis_name="c", subcore_axis_name="s")


@functools.partial(
    pl.kernel,
    mesh=_mesh,
    out_type=jax.ShapeDtypeStruct((B, K, D), jnp.float32),
    scratch_types=[
        pltpu.VMEM((BPW * K,), jnp.int32),
        pltpu.VMEM((2, K, D), jnp.float32),
        pltpu.SemaphoreType.DMA,
        pltpu.SemaphoreType.DMA,
        pltpu.SemaphoreType.DMA,
        pltpu.SemaphoreType.DMA,
    ],
)
def _gather_pe(table_hbm, idx_hbm, out_hbm, idx_v, rows_v,
               g0, g1, s0, s1):
    cid = lax.axis_index("c")
    sid = lax.axis_index("s")
    wid = sid * NC + cid
    b0 = wid * BPW

    pltpu.sync_copy(idx_hbm.at[pl.ds(b0 * K, BPW * K)], idx_v)

    gsems = (g0, g1)
    ssems = (s0, s1)

    def gather_cp(i, slot):
        return pltpu.make_async_copy(
            table_hbm.at[idx_v.at[pl.ds(i * K, K)]], rows_v.at[slot],
            gsems[slot])

    def store_cp(i, slot):
        return pltpu.make_async_copy(
            rows_v.at[slot], out_hbm.at[b0 + i], ssems[slot])

    gather_cp(0, 0).start()
    gather_cp(0, 0).wait()
    store_cp(0, 0).start()
    gather_cp(1, 1).start()

    def body(g, carry):
        for b in range(2):
            i = 1 + 2 * g + b
            slot = 1 - b
            gather_cp(i, slot).wait()
            store_cp(i, slot).start()
            store_cp(i - 1, 1 - slot).wait()
            gather_cp(i + 1, 1 - slot).start()
        return carry

    lax.fori_loop(0, (BPW - 2) // 2, body, 0)

    gather_cp(BPW - 1, 1).wait()
    store_cp(BPW - 1, 1).start()
    store_cp(BPW - 2, 0).wait()
    store_cp(BPW - 1, 1).wait()


def kernel(indices, pe):
    table = pe[0]
    idx = indices.astype(jnp.int32).reshape(-1)
    return _gather_pe(table, idx)


# speedup vs baseline: 9.8662x; 1.0425x over previous
import functools

import jax
import jax.numpy as jnp
from jax import lax
from jax.experimental import pallas as pl
from jax.experimental.pallas import tpu as pltpu
from jax.experimental.pallas import tpu_sc as plsc

B = 4096
K = 200
D = 128
V = 8192
NC = 2
NS = 16
NW = NC * NS
BPW = B // NW
VPT = V // NS

_mesh = plsc.VectorSubcoreMesh(core_axis_name="c", subcore_axis_name="s")


@functools.partial(
    pl.kernel,
    mesh=_mesh,
    out_type=jax.ShapeDtypeStruct((B, K, D), jnp.float32),
    scratch_types=[
        pltpu.VMEM((BPW * K,), jnp.int32),
        pltpu.VMEM((4, K, D), jnp.float32),
        pltpu.SemaphoreType.DMA,
        pltpu.SemaphoreType.DMA,
        pltpu.SemaphoreType.DMA,
        pltpu.SemaphoreType.DMA,
        pltpu.SemaphoreType.DMA,
        pltpu.SemaphoreType.DMA,
        pltpu.SemaphoreType.DMA,
        pltpu.SemaphoreType.DMA,
    ],
)
def _gather_pe(table_hbm, idx_hbm, out_hbm, idx_v, rows_v,
               g0, g1, g2, g3, s0, s1, s2, s3):
    cid = lax.axis_index("c")
    sid = lax.axis_index("s")
    wid = sid * NC + cid
    b0 = wid * BPW

    pltpu.sync_copy(idx_hbm.at[pl.ds(b0 * K, BPW * K)], idx_v)

    gsems = (g0, g1, g2, g3)
    ssems = (s0, s1, s2, s3)

    def gather_cp(i, slot):
        return pltpu.make_async_copy(
            table_hbm.at[idx_v.at[pl.ds(i * K, K)]], rows_v.at[slot],
            gsems[slot])

    def store_cp(i, slot):
        return pltpu.make_async_copy(
            rows_v.at[slot], out_hbm.at[b0 + i], ssems[slot])

    gather_cp(0, 0).start()
    gather_cp(1, 1).start()

    gather_cp(0, 0).wait()
    store_cp(0, 0).start()
    gather_cp(2, 2).start()

    gather_cp(1, 1).wait()
    store_cp(1, 1).start()
    gather_cp(3, 3).start()

    def body(g, carry):
        for b in range(4):
            i = 2 + 4 * g + b
            slot = (2 + b) % 4
            gather_cp(i, slot).wait()
            store_cp(i, slot).start()
            store_cp(i - 2, (slot - 2) % 4).wait()
            gather_cp(i + 2, (slot + 2) % 4).start()
        return carry

    lax.fori_loop(0, (BPW - 4) // 4, body, 0)

    gather_cp(BPW - 2, (BPW - 2) % 4).wait()
    store_cp(BPW - 2, (BPW - 2) % 4).start()
    store_cp(BPW - 4, (BPW - 4) % 4).wait()

    gather_cp(BPW - 1, (BPW - 1) % 4).wait()
    store_cp(BPW - 1, (BPW - 1) % 4).start()
    store_cp(BPW - 3, (BPW - 3) % 4).wait()
    store_cp(BPW - 2, (BPW - 2) % 4).wait()
    store_cp(BPW - 1, (BPW - 1) % 4).wait()


def kernel(indices, pe):
    table = pe[0]
    idx = indices.astype(jnp.int32).reshape(-1)
    return _gather_pe(table, idx)
